# baseline (speedup 1.0000x reference)
import functools

import jax
import jax.numpy as jnp
from jax import lax
from jax.experimental import pallas as pl
from jax.experimental.pallas import tpu as pltpu
from jax.experimental.pallas import tpu_sc as plsc

TW = 64
ROWS = 256
D = 64


@functools.lru_cache(maxsize=None)
def _make_gather(vocab: int, words: int, batch: int):
    info = plsc.get_sparse_core_info()
    nc, ns = info.num_cores, info.num_subcores
    nw = nc * ns
    b_per_w = batch // nw
    n_steps = words * b_per_w // ROWS
    half = b_per_w // ROWS
    assert b_per_w % ROWS == 0 and batch == nw * b_per_w

    mesh = plsc.VectorSubcoreMesh(core_axis_name="c", subcore_axis_name="s")

    @functools.partial(
        pl.kernel,
        mesh=mesh,
        out_type=jax.ShapeDtypeStruct((words, D, batch), jnp.float32),
        scratch_types=[
            pltpu.VMEM((words, b_per_w), jnp.int32),
            pltpu.VMEM((2, ROWS, TW), jnp.float32),
            pltpu.VMEM((2, D, ROWS), jnp.float32),
            pltpu.SemaphoreType.DMA,
            pltpu.SemaphoreType.DMA,
            pltpu.SemaphoreType.DMA,
        ],
        compiler_params=pltpu.CompilerParams(
            use_tc_tiling_on_sc=False, needs_layout_passes=False
        ),
    )
    def gather_kernel(xt_hbm, tp_hbm, out_hbm, idx_v, bufA, bufT, isem, gsem, ssem):
        wid = lax.axis_index("s") * nc + lax.axis_index("c")
        b0 = wid * b_per_w

        pltpu.async_copy(xt_hbm.at[:, pl.ds(b0, b_per_w)], idx_v, isem).wait()

        iota16 = lax.iota(jnp.int32, 16)
        zeros16 = jnp.zeros((16,), jnp.int32)

        def fire_gathers(t, p):
            w = lax.div(t, half)
            col0 = lax.rem(t, half) * ROWS
            for k in range(ROWS // 128):
                pltpu.async_copy(
                    tp_hbm.at[idx_v.at[w, pl.ds(col0 + k * 128, 128)]],
                    bufA.at[p, pl.ds(k * 128, 128)],
                    gsem,
                )

        def drain_gathers(p):
            for k in range(ROWS // 128):
                pltpu.make_async_copy(
                    tp_hbm.at[pl.ds(0, 128)],
                    bufA.at[p, pl.ds(k * 128, 128)],
                    gsem,
                ).wait()

        def fire_store(t, p):
            w = lax.div(t, half)
            col0 = lax.rem(t, half) * ROWS
            pltpu.make_async_copy(
                bufT.at[p],
                out_hbm.at[w, :, pl.ds(b0 + col0, ROWS)],
                ssem,
            ).start()

        def drain_store(p):
            pltpu.make_async_copy(
                bufT.at[p],
                out_hbm.at[0, :, pl.ds(b0, ROWS)],
                ssem,
            ).wait()

        fire_gathers(0, 0)

        def loop_body(t2, carry):
            for p in range(2):
                t = 2 * t2 + p
                drain_gathers(p)

                @pl.when(t + 1 < n_steps)
                def _():
                    fire_gathers(t + 1, 1 - p)

                @pl.when(t >= 2)
                def _():
                    drain_store(p)

                @plsc.parallel_loop(0, ROWS, unroll=16)
                def _(r):
                    rvec = zeros16 + r
                    for seg in range(TW // 16):
                        dvec = iota16 + (16 * seg)
                        val = bufA[p, r, pl.ds(seg * 16, 16)]
                        plsc.store_scatter(bufT.at[p], [dvec, rvec], val)
                fire_store(t, p)
            return carry

        lax.fori_loop(0, n_steps // 2, loop_body, 0)
        drain_store(0)
        drain_store(1)

    return gather_kernel, nw


def kernel(x, table):
    vocab, dim = table.shape
    batch, words = x.shape
    gather_kernel, nw = _make_gather(vocab, words, batch)
    xt = x.astype(jnp.int32).T
    out = gather_kernel(xt, table)
    return jnp.transpose(out, (2, 0, 1))


# speedup vs baseline: 1.2193x; 1.2193x over previous
import functools

import jax
import jax.numpy as jnp
from jax import lax
from jax.experimental import pallas as pl
from jax.experimental.pallas import tpu as pltpu
from jax.experimental.pallas import tpu_sc as plsc

NBUF = 8
AHEAD = 6


@functools.lru_cache(maxsize=None)
def _make_gather(vocab: int, words: int, dim: int, batch: int):
    info = plsc.get_sparse_core_info()
    nc, ns = info.num_cores, info.num_subcores
    nw = nc * ns
    b_per_w = batch // nw
    assert batch == nw * b_per_w

    mesh = plsc.VectorSubcoreMesh(core_axis_name="c", subcore_axis_name="s")

    @functools.partial(
        pl.kernel,
        mesh=mesh,
        out_type=jax.ShapeDtypeStruct((batch, words, dim), jnp.float32),
        scratch_types=[
            pltpu.VMEM((b_per_w, words), jnp.int32),
            pltpu.VMEM((NBUF, words, dim), jnp.float32),
            pltpu.SemaphoreType.DMA,
            pltpu.SemaphoreType.DMA,
            pltpu.SemaphoreType.DMA,
        ],
        compiler_params=pltpu.CompilerParams(use_tc_tiling_on_sc=False),
    )
    def gather_kernel(x_hbm, table_hbm, out_hbm, idx_v, bufs, isem, gsem, ssem):
        wid = lax.axis_index("s") * nc + lax.axis_index("c")
        i0 = wid * b_per_w

        pltpu.async_copy(x_hbm.at[pl.ds(i0, b_per_w)], idx_v, isem).wait()

        def fire_gather(j):
            m = lax.rem(j, NBUF)
            pltpu.async_copy(table_hbm.at[idx_v.at[j]], bufs.at[m], gsem)

        def drain_gather(j):
            m = lax.rem(j, NBUF)
            pltpu.make_async_copy(
                table_hbm.at[pl.ds(0, words)], bufs.at[m], gsem
            ).wait()

        def fire_store(j):
            m = lax.rem(j, NBUF)
            pltpu.make_async_copy(bufs.at[m], out_hbm.at[i0 + j], ssem).start()

        def drain_store(j):
            m = lax.rem(j, NBUF)
            pltpu.make_async_copy(bufs.at[m], out_hbm.at[i0], ssem).wait()

        for j in range(AHEAD):
            fire_gather(j)

        def step(j, carry):
            drain_gather(j)

            @pl.when(j + AHEAD < b_per_w)
            def _():
                @pl.when(j + AHEAD >= NBUF)
                def _():
                    drain_store(j + AHEAD - NBUF)

                fire_gather(j + AHEAD)

            fire_store(j)
            return carry

        lax.fori_loop(0, b_per_w, step, 0)
        for j in range(b_per_w - NBUF, b_per_w):
            drain_store(j)

    return gather_kernel, nw


def kernel(x, table):
    vocab, dim = table.shape
    batch, words = x.shape
    gather_kernel, nw = _make_gather(vocab, words, dim, batch)
    return gather_kernel(x.astype(jnp.int32), table)


# speedup vs baseline: 1.2220x; 1.0022x over previous
import functools

import jax
import jax.numpy as jnp
from jax import lax
from jax.experimental import pallas as pl
from jax.experimental.pallas import tpu as pltpu
from jax.experimental.pallas import tpu_sc as plsc

NBUF = 12
AHEAD = 10


@functools.lru_cache(maxsize=None)
def _make_gather(vocab: int, words: int, dim: int, batch: int):
    info = plsc.get_sparse_core_info()
    nc, ns = info.num_cores, info.num_subcores
    nw = nc * ns
    b_per_w = batch // nw
    assert batch == nw * b_per_w

    mesh = plsc.VectorSubcoreMesh(core_axis_name="c", subcore_axis_name="s")

    @functools.partial(
        pl.kernel,
        mesh=mesh,
        out_type=jax.ShapeDtypeStruct((batch, words, dim), jnp.float32),
        scratch_types=[
            pltpu.VMEM((b_per_w, words), jnp.int32),
            pltpu.VMEM((NBUF, words, dim), jnp.float32),
            pltpu.SemaphoreType.DMA,
            pltpu.SemaphoreType.DMA,
            pltpu.SemaphoreType.DMA,
        ],
        compiler_params=pltpu.CompilerParams(use_tc_tiling_on_sc=False),
    )
    def gather_kernel(x_hbm, table_hbm, out_hbm, idx_v, bufs, isem, gsem, ssem):
        wid = lax.axis_index("s") * nc + lax.axis_index("c")
        i0 = wid * b_per_w

        pltpu.async_copy(x_hbm.at[pl.ds(i0, b_per_w)], idx_v, isem).wait()

        def fire_gather(j):
            m = lax.rem(j, NBUF)
            pltpu.async_copy(table_hbm.at[idx_v.at[j]], bufs.at[m], gsem)

        def drain_gather(j):
            m = lax.rem(j, NBUF)
            pltpu.make_async_copy(
                table_hbm.at[pl.ds(0, words)], bufs.at[m], gsem
            ).wait()

        def fire_store(j):
            m = lax.rem(j, NBUF)
            pltpu.make_async_copy(bufs.at[m], out_hbm.at[i0 + j], ssem).start()

        def drain_store(j):
            m = lax.rem(j, NBUF)
            pltpu.make_async_copy(bufs.at[m], out_hbm.at[i0], ssem).wait()

        for j in range(AHEAD):
            fire_gather(j)

        def step(j, carry):
            drain_gather(j)

            @pl.when(j + AHEAD < b_per_w)
            def _():
                @pl.when(j + AHEAD >= NBUF)
                def _():
                    drain_store(j + AHEAD - NBUF)

                fire_gather(j + AHEAD)

            fire_store(j)
            return carry

        lax.fori_loop(0, b_per_w, step, 0)
        for j in range(b_per_w - NBUF, b_per_w):
            drain_store(j)

    return gather_kernel, nw


def kernel(x, table):
    vocab, dim = table.shape
    batch, words = x.shape
    gather_kernel, nw = _make_gather(vocab, words, dim, batch)
    return gather_kernel(x.astype(jnp.int32), table)
